# Initial kernel scaffold; baseline (speedup 1.0000x reference)
import jax
import jax.numpy as jnp
from jax.experimental import pallas as pl


def kernel(images, predictions):
    raise NotImplementedError("write your pallas kernel here")


# speedup vs baseline: 1.0727x; 1.0727x over previous
import numpy as np
import jax
import jax.numpy as jnp
from jax.experimental import pallas as pl
from jax.experimental.pallas import tpu as pltpu

_IMG = 640
_B = 8
_C = 80
_N = 76725
_NPAD = 76800
_K = 100
_IOU_TH = 0.5
_SCORE_TH = 0.05
_PAD_NEG = -1e9


def _anchors_np():
    ratios = [0.5, 1.0, 2.0]
    scales = [2.0 ** 0, 2.0 ** (1.0 / 3.0), 2.0 ** (2.0 / 3.0)]
    strides = [8, 16, 32, 64, 128]
    areas = [32.0 ** 2, 64.0 ** 2, 128.0 ** 2, 256.0 ** 2, 512.0 ** 2]
    out = []
    for lvl in range(5):
        fh = int(np.ceil(_IMG / strides[lvl]))
        fw = int(np.ceil(_IMG / strides[lvl]))
        dims = []
        for r in ratios:
            ah = np.sqrt(areas[lvl] / r)
            aw = areas[lvl] / ah
            for s in scales:
                dims.append([s * aw, s * ah])
        dims = np.array(dims, np.float32)
        rx = (np.arange(fw, dtype=np.float32) + 0.5) * strides[lvl]
        ry = (np.arange(fh, dtype=np.float32) + 0.5) * strides[lvl]
        cx, cy = np.meshgrid(rx, ry)
        centers = np.stack([cx, cy], axis=-1)[:, :, None, :]
        centers = np.broadcast_to(centers, (fh, fw, 9, 2))
        d = np.broadcast_to(dims[None, None], (fh, fw, 9, 2))
        out.append(np.concatenate([centers, d], axis=-1).reshape(-1, 4))
    a = np.concatenate(out, axis=0)
    pad = np.zeros((_NPAD - _N, 4), np.float32)
    return np.concatenate([a, pad], axis=0)


_ANCHORS = _anchors_np()


def _decode_kernel(p_ref, a_ref, s_ref, b_ref):
    p = p_ref[0]
    a = a_ref[...]
    logits = p[:, 4:84]
    s = jax.nn.sigmoid(logits)
    s_ref[0] = jnp.where(s > _SCORE_TH, s, -1.0)
    xy = (p[:, 0:2] * 0.1) * a[:, 2:4] + a[:, 0:2]
    wh = jnp.exp(p[:, 2:4] * 0.2) * a[:, 2:4]
    b_ref[0] = jnp.concatenate([xy - wh * 0.5, xy + wh * 0.5], axis=1)


def _decode_scores_boxes(preds_pad, anchors):
    R = 7680
    grid = (_B, _NPAD // R)
    return pl.pallas_call(
        _decode_kernel,
        grid=grid,
        in_specs=[
            pl.BlockSpec((1, R, 84), lambda b, n: (b, n, 0)),
            pl.BlockSpec((R, 4), lambda b, n: (n, 0)),
        ],
        out_specs=[
            pl.BlockSpec((1, R, _C), lambda b, n: (b, n, 0)),
            pl.BlockSpec((1, R, 4), lambda b, n: (b, n, 0)),
        ],
        out_shape=[
            jax.ShapeDtypeStruct((_B, _NPAD, _C), jnp.float32),
            jax.ShapeDtypeStruct((_B, _NPAD, 4), jnp.float32),
        ],
        compiler_params=pltpu.CompilerParams(
            dimension_semantics=("parallel", "parallel"),
        ),
    )(preds_pad, anchors)


def _iou_one_vs_all(box, boxes):
    lt = jnp.maximum(box[:2], boxes[:, :2])
    rb = jnp.minimum(box[2:], boxes[:, 2:])
    wh = jnp.clip(rb - lt, 0.0)
    inter = wh[:, 0] * wh[:, 1]
    a1 = (box[2] - box[0]) * (box[3] - box[1])
    a2 = (boxes[:, 2] - boxes[:, 0]) * (boxes[:, 3] - boxes[:, 1])
    return inter / (a1 + a2 - inter + 1e-8)


def _nms_single_class(b, sk):
    arange = jnp.arange(_K)

    def body(i, keep):
        ious = _iou_one_vs_all(b[i], b)
        sup = (ious > _IOU_TH) & (arange > i) & keep[i]
        return keep & (~sup)

    keep = jax.lax.fori_loop(0, _K, body, sk > 0.0)
    return jnp.where(keep, sk, -1.0)


def _per_image(tb, ts):
    cs = jax.vmap(_nms_single_class)(tb, ts)
    cls_ids = jnp.broadcast_to(
        jnp.arange(_C, dtype=jnp.float32)[:, None], (_C, _K))
    fb = tb.reshape(-1, 4)
    fs = cs.reshape(-1)
    fc = cls_ids.reshape(-1)
    top_s, top_i = jax.lax.top_k(fs, _K)
    ob = fb[top_i]
    oc = fc[top_i]
    valid = jnp.sum(top_s > 0.0).astype(jnp.int32)
    good = top_s > 0.0
    return (jnp.where(good[:, None], ob, 0.0),
            jnp.maximum(top_s, 0.0),
            jnp.where(good, oc, 0.0),
            valid)


def kernel(images, predictions):
    preds = predictions.reshape(_B, _N, _C + 4)
    preds_pad = jnp.pad(preds, ((0, 0), (0, _NPAD - _N), (0, 0)),
                        constant_values=_PAD_NEG)
    anchors = jnp.asarray(_ANCHORS)
    scores, boxes = _decode_scores_boxes(preds_pad, anchors)

    sT = scores.transpose(0, 2, 1)
    ts, idx = jax.lax.top_k(sT, _K)
    tb = jnp.take_along_axis(boxes[:, None], idx[..., None], axis=2)

    return jax.vmap(_per_image)(tb, ts)


# speedup vs baseline: 6.0129x; 5.6052x over previous
import numpy as np
import jax
import jax.numpy as jnp
from jax.experimental import pallas as pl
from jax.experimental.pallas import tpu as pltpu

_IMG = 640
_B = 8
_C = 80
_N = 76725
_NPAD = 76800
_K = 100
_IOU_TH = 0.5
_SCORE_TH = 0.05
_PAD_NEG = -1e9
_G = 4
_L = _G * _C


def _anchors_np():
    ratios = [0.5, 1.0, 2.0]
    scales = [2.0 ** 0, 2.0 ** (1.0 / 3.0), 2.0 ** (2.0 / 3.0)]
    strides = [8, 16, 32, 64, 128]
    areas = [32.0 ** 2, 64.0 ** 2, 128.0 ** 2, 256.0 ** 2, 512.0 ** 2]
    out = []
    for lvl in range(5):
        fh = int(np.ceil(_IMG / strides[lvl]))
        fw = int(np.ceil(_IMG / strides[lvl]))
        dims = []
        for r in ratios:
            ah = np.sqrt(areas[lvl] / r)
            aw = areas[lvl] / ah
            for s in scales:
                dims.append([s * aw, s * ah])
        dims = np.array(dims, np.float32)
        rx = (np.arange(fw, dtype=np.float32) + 0.5) * strides[lvl]
        ry = (np.arange(fh, dtype=np.float32) + 0.5) * strides[lvl]
        cx, cy = np.meshgrid(rx, ry)
        centers = np.stack([cx, cy], axis=-1)[:, :, None, :]
        centers = np.broadcast_to(centers, (fh, fw, 9, 2))
        d = np.broadcast_to(dims[None, None], (fh, fw, 9, 2))
        out.append(np.concatenate([centers, d], axis=-1).reshape(-1, 4))
    a = np.concatenate(out, axis=0)
    pad = np.zeros((_NPAD - _N, 4), np.float32)
    return np.concatenate([a, pad], axis=0)


_ANCHORS = _anchors_np()


def _decode_kernel(p_ref, a_ref, s_ref, b_ref):
    p = p_ref[0]
    a = a_ref[...]
    logits = p[:, 4:84]
    s = jax.nn.sigmoid(logits)
    s_ref[0] = jnp.where(s > _SCORE_TH, s, -1.0).T
    xy = (p[:, 0:2] * 0.1) * a[:, 2:4] + a[:, 0:2]
    wh = jnp.exp(p[:, 2:4] * 0.2) * a[:, 2:4]
    b_ref[0] = jnp.concatenate([xy - wh * 0.5, xy + wh * 0.5], axis=1)


def _decode_scores_boxes(preds_pad, anchors):
    R = 7680
    grid = (_B, _NPAD // R)
    return pl.pallas_call(
        _decode_kernel,
        grid=grid,
        in_specs=[
            pl.BlockSpec((1, R, 84), lambda b, n: (b, n, 0)),
            pl.BlockSpec((R, 4), lambda b, n: (n, 0)),
        ],
        out_specs=[
            pl.BlockSpec((1, _C, R), lambda b, n: (b, 0, n)),
            pl.BlockSpec((1, R, 4), lambda b, n: (b, n, 0)),
        ],
        out_shape=[
            jax.ShapeDtypeStruct((_B, _C, _NPAD), jnp.float32),
            jax.ShapeDtypeStruct((_B, _NPAD, 4), jnp.float32),
        ],
        compiler_params=pltpu.CompilerParams(
            dimension_semantics=("parallel", "parallel"),
        ),
    )(preds_pad, anchors)


_CH = 40


def _topk_kernel(s_ref, out_ref):
    li = jax.lax.broadcasted_iota(jnp.int32, (1, _CH, 128), 2)

    def body(t, am_prev):
        ii = jax.lax.broadcasted_iota(jnp.int32, (_CH, _NPAD), 1)
        sp = jnp.where(ii == am_prev, -2.0, s_ref[0, 0])
        s_ref[0, 0] = sp
        m = jnp.max(sp, axis=1, keepdims=True)
        cand = jnp.where(sp == m, ii, jnp.int32(1 << 30))
        am = jnp.min(cand, axis=1, keepdims=True)
        amf = am.astype(jnp.float32)
        row = jnp.where(li == 0, m[None], jnp.where(li == 1, amf[None], 0.0))
        out_ref[0, pl.ds(t, 1), :, :] = row
        return am

    jax.lax.fori_loop(0, _K, body,
                      jnp.full((_CH, 1), -1, jnp.int32))


def _topk(sT4):
    nh = _C // _CH
    out = pl.pallas_call(
        _topk_kernel,
        grid=(_B, nh),
        in_specs=[pl.BlockSpec((1, 1, _CH, _NPAD), lambda b, h: (b, h, 0, 0))],
        out_specs=pl.BlockSpec((1, _K, _CH, 128), lambda b, h: (b * nh + h, 0, 0, 0)),
        out_shape=jax.ShapeDtypeStruct((_B * nh, _K, _CH, 128), jnp.float32),
        compiler_params=pltpu.CompilerParams(
            dimension_semantics=("parallel", "parallel"),
            vmem_limit_bytes=56 * 1024 * 1024,
        ),
    )(sT4)
    return out


def _nms_kernel(s_ref, x1_ref, y1_ref, x2_ref, y2_ref,
                out_ref, keep_ref, fs_ref, a2_ref):
    s0 = s_ref[0, :, 0, :]
    x1 = x1_ref[0, :, 0, :]
    y1 = y1_ref[0, :, 0, :]
    x2 = x2_ref[0, :, 0, :]
    y2 = y2_ref[0, :, 0, :]

    keep_ref[:, 0, :] = jnp.where(s0 > 0.0, 1.0, 0.0)
    a2_ref[:, 0, :] = (x2 - x1) * (y2 - y1)
    j_gt = jax.lax.broadcasted_iota(jnp.int32, (_K, 1), 0)

    def nms_body(i, _):
        bx1 = x1_ref[0, pl.ds(i, 1), 0, :]
        by1 = y1_ref[0, pl.ds(i, 1), 0, :]
        bx2 = x2_ref[0, pl.ds(i, 1), 0, :]
        by2 = y2_ref[0, pl.ds(i, 1), 0, :]
        ki = keep_ref[pl.ds(i, 1), 0, :]
        w = jnp.clip(jnp.minimum(bx2, x2_ref[0, :, 0, :])
                     - jnp.maximum(bx1, x1_ref[0, :, 0, :]), 0.0)
        h = jnp.clip(jnp.minimum(by2, y2_ref[0, :, 0, :])
                     - jnp.maximum(by1, y1_ref[0, :, 0, :]), 0.0)
        inter = w * h
        a1 = (bx2 - bx1) * (by2 - by1)
        iou = inter / (a1 + a2_ref[:, 0, :] - inter + 1e-8)
        sup = (iou > _IOU_TH) & (j_gt > i) & (ki > 0.0)
        keep_ref[:, 0, :] = jnp.where(sup, 0.0, keep_ref[:, 0, :])
        return 0

    jax.lax.fori_loop(0, _K, nms_body, 0)

    fs = jnp.where(keep_ref[:, 0, :] > 0.0, s0, -1.0)

    def to3(v):
        return jnp.concatenate(
            [v[:, g * _C:(g + 1) * _C] for g in range(_G)], axis=0
        ).reshape(_G, _K, _C)

    fs_ref[...] = to3(fs)
    x13 = to3(x1)
    y13 = to3(y1)
    x23 = to3(x2)
    y23 = to3(y2)

    ck = (jax.lax.broadcasted_iota(jnp.int32, (_G, _K, _C), 2) * _K
          + jax.lax.broadcasted_iota(jnp.int32, (_G, _K, _C), 1)
          ).astype(jnp.float32)
    li = jax.lax.broadcasted_iota(jnp.int32, (_G, 1, 128), 2)

    def sel_body(t, _):
        f = fs_ref[...]
        m = jnp.max(f, axis=(1, 2), keepdims=True)
        cand = jnp.where(f == m, ck, 1e9)
        am = jnp.min(cand, axis=(1, 2), keepdims=True)
        oh = ck == am
        cls = jnp.floor((am + 0.5) * 0.01)
        bx1 = jnp.sum(jnp.where(oh, x13, 0.0), axis=(1, 2), keepdims=True)
        by1 = jnp.sum(jnp.where(oh, y13, 0.0), axis=(1, 2), keepdims=True)
        bx2 = jnp.sum(jnp.where(oh, x23, 0.0), axis=(1, 2), keepdims=True)
        by2 = jnp.sum(jnp.where(oh, y23, 0.0), axis=(1, 2), keepdims=True)
        good = m > 0.0
        z = jnp.zeros_like(m)
        row = jnp.where(li == 0, jnp.maximum(m, 0.0),
              jnp.where(li == 1, jnp.where(good, cls, z),
              jnp.where(li == 2, jnp.where(good, bx1, z),
              jnp.where(li == 3, jnp.where(good, by1, z),
              jnp.where(li == 4, jnp.where(good, bx2, z),
              jnp.where(li == 5, jnp.where(good, by2, z), z))))))
        out_ref[0, :, pl.ds(t, 1), 0, :] = row
        fs_ref[...] = jnp.where(oh, -2.0, f)
        return 0

    jax.lax.fori_loop(0, _K, sel_body, 0)


def _nms_select(s_kl, x1, y1, x2, y2):
    nprog = _B // _G
    out = pl.pallas_call(
        _nms_kernel,
        grid=(nprog,),
        in_specs=[pl.BlockSpec((1, _K, 1, _L), lambda g: (g, 0, 0, 0))] * 5,
        out_specs=pl.BlockSpec((1, _G, _K, 1, 128), lambda g: (g, 0, 0, 0, 0)),
        out_shape=jax.ShapeDtypeStruct((nprog, _G, _K, 1, 128), jnp.float32),
        scratch_shapes=[
            pltpu.VMEM((_K, 1, _L), jnp.float32),
            pltpu.VMEM((_G, _K, _C), jnp.float32),
            pltpu.VMEM((_K, 1, _L), jnp.float32),
        ],
        compiler_params=pltpu.CompilerParams(
            dimension_semantics=("parallel",),
        ),
    )(s_kl, x1, y1, x2, y2)
    return out


def kernel(images, predictions):
    preds = predictions.reshape(_B, _N, _C + 4)
    preds_pad = jnp.pad(preds, ((0, 0), (0, _NPAD - _N), (0, 0)),
                        constant_values=_PAD_NEG)
    anchors = jnp.asarray(_ANCHORS)
    sT, boxes = _decode_scores_boxes(preds_pad, anchors)
    nh = _C // _CH
    tk = _topk(sT.reshape(_B, nh, _CH, _NPAD))
    ts = (tk[:, :, :, 0].reshape(_B, nh, _K, _CH)
          .transpose(0, 1, 3, 2).reshape(_B, _C, _K))
    idx = (tk[:, :, :, 1].reshape(_B, nh, _K, _CH)
           .transpose(0, 1, 3, 2).reshape(_B, _C, _K)).astype(jnp.int32)
    tb = jnp.take_along_axis(boxes[:, None], idx[..., None], axis=2)

    nprog = _B // _G

    def to_kl(v):
        return (v.reshape(nprog, _G, _C, _K)
                 .transpose(0, 3, 1, 2)
                 .reshape(nprog, _K, 1, _L))

    s_kl = to_kl(ts)
    x1 = to_kl(tb[..., 0])
    y1 = to_kl(tb[..., 1])
    x2 = to_kl(tb[..., 2])
    y2 = to_kl(tb[..., 3])

    raw = _nms_select(s_kl, x1, y1, x2, y2)
    raw = raw.reshape(_B, _K, 128)
    out_scores = raw[:, :, 0]
    out_classes = raw[:, :, 1]
    out_boxes = raw[:, :, 2:6]
    valid = jnp.sum(out_scores > 0.0, axis=1).astype(jnp.int32)
    return out_boxes, out_scores, out_classes, valid


# speedup vs baseline: 6.6682x; 1.1090x over previous
import numpy as np
import jax
import jax.numpy as jnp
from jax.experimental import pallas as pl
from jax.experimental.pallas import tpu as pltpu

_IMG = 640
_B = 8
_C = 80
_N = 76725
_NPAD = 76800
_K = 100
_IOU_TH = 0.5
_SCORE_TH = 0.05
_PAD_NEG = -1e9
_G = 4
_L = _G * _C


def _anchors_np():
    ratios = [0.5, 1.0, 2.0]
    scales = [2.0 ** 0, 2.0 ** (1.0 / 3.0), 2.0 ** (2.0 / 3.0)]
    strides = [8, 16, 32, 64, 128]
    areas = [32.0 ** 2, 64.0 ** 2, 128.0 ** 2, 256.0 ** 2, 512.0 ** 2]
    out = []
    for lvl in range(5):
        fh = int(np.ceil(_IMG / strides[lvl]))
        fw = int(np.ceil(_IMG / strides[lvl]))
        dims = []
        for r in ratios:
            ah = np.sqrt(areas[lvl] / r)
            aw = areas[lvl] / ah
            for s in scales:
                dims.append([s * aw, s * ah])
        dims = np.array(dims, np.float32)
        rx = (np.arange(fw, dtype=np.float32) + 0.5) * strides[lvl]
        ry = (np.arange(fh, dtype=np.float32) + 0.5) * strides[lvl]
        cx, cy = np.meshgrid(rx, ry)
        centers = np.stack([cx, cy], axis=-1)[:, :, None, :]
        centers = np.broadcast_to(centers, (fh, fw, 9, 2))
        d = np.broadcast_to(dims[None, None], (fh, fw, 9, 2))
        out.append(np.concatenate([centers, d], axis=-1).reshape(-1, 4))
    a = np.concatenate(out, axis=0)
    pad = np.zeros((_NPAD - _N, 4), np.float32)
    return np.concatenate([a, pad], axis=0)


_ANCHORS = _anchors_np()


_DR = 7680


def _decode_kernel(p_ref, a_ref, s_ref, b_ref):
    p = p_ref[0]
    a = a_ref[...]
    n = pl.program_id(1)
    row = jax.lax.broadcasted_iota(jnp.int32, (_DR, 1), 0) + n * _DR
    ok = row < _N
    logits = p[:, 4:84]
    s = jax.nn.sigmoid(logits)
    s_ref[0] = jnp.where(ok & (s > _SCORE_TH), s, -1.0).T
    xy = (p[:, 0:2] * 0.1) * a[:, 2:4] + a[:, 0:2]
    wh = jnp.exp(jnp.where(ok, p[:, 2:4], 0.0) * 0.2) * a[:, 2:4]
    b_ref[0] = jnp.concatenate([xy - wh * 0.5, xy + wh * 0.5], axis=1)


def _decode_scores_boxes(preds, anchors):
    R = _DR
    grid = (_B, _NPAD // R)
    return pl.pallas_call(
        _decode_kernel,
        grid=grid,
        in_specs=[
            pl.BlockSpec((1, R, 84), lambda b, n: (b, n, 0)),
            pl.BlockSpec((R, 4), lambda b, n: (n, 0)),
        ],
        out_specs=[
            pl.BlockSpec((1, _C, R), lambda b, n: (b, 0, n)),
            pl.BlockSpec((1, R, 4), lambda b, n: (b, n, 0)),
        ],
        out_shape=[
            jax.ShapeDtypeStruct((_B, _C, _NPAD), jnp.float32),
            jax.ShapeDtypeStruct((_B, _NPAD, 4), jnp.float32),
        ],
        compiler_params=pltpu.CompilerParams(
            dimension_semantics=("parallel", "parallel"),
        ),
    )(preds, anchors)


_CH = 40


def _topk_kernel(s_ref, out_ref):
    li = jax.lax.broadcasted_iota(jnp.int32, (1, _CH, 128), 2)

    def body(t, am_prev):
        ii = jax.lax.broadcasted_iota(jnp.int32, (_CH, _NPAD), 1)
        sp = jnp.where(ii == am_prev, -2.0, s_ref[0, 0])
        s_ref[0, 0] = sp
        m = jnp.max(sp, axis=1, keepdims=True)
        cand = jnp.where(sp == m, ii, jnp.int32(1 << 30))
        am = jnp.min(cand, axis=1, keepdims=True)
        amf = am.astype(jnp.float32)
        row = jnp.where(li == 0, m[None], jnp.where(li == 1, amf[None], 0.0))
        out_ref[0, pl.ds(t, 1), :, :] = row
        return am

    jax.lax.fori_loop(0, _K, body,
                      jnp.full((_CH, 1), -1, jnp.int32))


def _topk(sT4):
    nh = _C // _CH
    out = pl.pallas_call(
        _topk_kernel,
        grid=(_B, nh),
        in_specs=[pl.BlockSpec((1, 1, _CH, _NPAD), lambda b, h: (b, h, 0, 0))],
        out_specs=pl.BlockSpec((1, _K, _CH, 128), lambda b, h: (b * nh + h, 0, 0, 0)),
        out_shape=jax.ShapeDtypeStruct((_B * nh, _K, _CH, 128), jnp.float32),
        compiler_params=pltpu.CompilerParams(
            dimension_semantics=("parallel", "parallel"),
            vmem_limit_bytes=56 * 1024 * 1024,
        ),
    )(sT4)
    return out


def _nms_kernel(s_ref, x1_ref, y1_ref, x2_ref, y2_ref,
                out_ref, keep_ref, fs_ref, a2_ref):
    s0 = s_ref[0, :, 0, :]
    x1 = x1_ref[0, :, 0, :]
    y1 = y1_ref[0, :, 0, :]
    x2 = x2_ref[0, :, 0, :]
    y2 = y2_ref[0, :, 0, :]

    keep_ref[:, 0, :] = jnp.where(s0 > 0.0, 1.0, 0.0)
    a2_ref[:, 0, :] = (x2 - x1) * (y2 - y1)
    j_gt = jax.lax.broadcasted_iota(jnp.int32, (_K, 1), 0)

    def nms_body(i, _):
        bx1 = x1_ref[0, pl.ds(i, 1), 0, :]
        by1 = y1_ref[0, pl.ds(i, 1), 0, :]
        bx2 = x2_ref[0, pl.ds(i, 1), 0, :]
        by2 = y2_ref[0, pl.ds(i, 1), 0, :]
        ki = keep_ref[pl.ds(i, 1), 0, :]
        w = jnp.clip(jnp.minimum(bx2, x2_ref[0, :, 0, :])
                     - jnp.maximum(bx1, x1_ref[0, :, 0, :]), 0.0)
        h = jnp.clip(jnp.minimum(by2, y2_ref[0, :, 0, :])
                     - jnp.maximum(by1, y1_ref[0, :, 0, :]), 0.0)
        inter = w * h
        a1 = (bx2 - bx1) * (by2 - by1)
        iou = inter / (a1 + a2_ref[:, 0, :] - inter + 1e-8)
        sup = (iou > _IOU_TH) & (j_gt > i) & (ki > 0.0)
        keep_ref[:, 0, :] = jnp.where(sup, 0.0, keep_ref[:, 0, :])
        return 0

    jax.lax.fori_loop(0, _K, nms_body, 0)

    fs = jnp.where(keep_ref[:, 0, :] > 0.0, s0, -1.0)

    def to3(v):
        return jnp.concatenate(
            [v[:, g * _C:(g + 1) * _C] for g in range(_G)], axis=0
        ).reshape(_G, _K, _C)

    fs_ref[...] = to3(fs)
    x13 = to3(x1)
    y13 = to3(y1)
    x23 = to3(x2)
    y23 = to3(y2)

    ck = (jax.lax.broadcasted_iota(jnp.int32, (_G, _K, _C), 2) * _K
          + jax.lax.broadcasted_iota(jnp.int32, (_G, _K, _C), 1)
          ).astype(jnp.float32)
    li = jax.lax.broadcasted_iota(jnp.int32, (_G, 1, 128), 2)

    def sel_body(t, _):
        f = fs_ref[...]
        m = jnp.max(f, axis=(1, 2), keepdims=True)
        cand = jnp.where(f == m, ck, 1e9)
        am = jnp.min(cand, axis=(1, 2), keepdims=True)
        oh = ck == am
        cls = jnp.floor((am + 0.5) * 0.01)
        bx1 = jnp.sum(jnp.where(oh, x13, 0.0), axis=(1, 2), keepdims=True)
        by1 = jnp.sum(jnp.where(oh, y13, 0.0), axis=(1, 2), keepdims=True)
        bx2 = jnp.sum(jnp.where(oh, x23, 0.0), axis=(1, 2), keepdims=True)
        by2 = jnp.sum(jnp.where(oh, y23, 0.0), axis=(1, 2), keepdims=True)
        good = m > 0.0
        z = jnp.zeros_like(m)
        row = jnp.where(li == 0, jnp.maximum(m, 0.0),
              jnp.where(li == 1, jnp.where(good, cls, z),
              jnp.where(li == 2, jnp.where(good, bx1, z),
              jnp.where(li == 3, jnp.where(good, by1, z),
              jnp.where(li == 4, jnp.where(good, bx2, z),
              jnp.where(li == 5, jnp.where(good, by2, z), z))))))
        out_ref[0, :, pl.ds(t, 1), 0, :] = row
        fs_ref[...] = jnp.where(oh, -2.0, f)
        return 0

    jax.lax.fori_loop(0, _K, sel_body, 0)


def _nms_select(s_kl, x1, y1, x2, y2):
    nprog = _B // _G
    out = pl.pallas_call(
        _nms_kernel,
        grid=(nprog,),
        in_specs=[pl.BlockSpec((1, _K, 1, _L), lambda g: (g, 0, 0, 0))] * 5,
        out_specs=pl.BlockSpec((1, _G, _K, 1, 128), lambda g: (g, 0, 0, 0, 0)),
        out_shape=jax.ShapeDtypeStruct((nprog, _G, _K, 1, 128), jnp.float32),
        scratch_shapes=[
            pltpu.VMEM((_K, 1, _L), jnp.float32),
            pltpu.VMEM((_G, _K, _C), jnp.float32),
            pltpu.VMEM((_K, 1, _L), jnp.float32),
        ],
        compiler_params=pltpu.CompilerParams(
            dimension_semantics=("parallel",),
        ),
    )(s_kl, x1, y1, x2, y2)
    return out


def kernel(images, predictions):
    preds = predictions.reshape(_B, _N, _C + 4)
    anchors = jnp.asarray(_ANCHORS)
    sT, boxes = _decode_scores_boxes(preds, anchors)
    nh = _C // _CH
    tk = _topk(sT.reshape(_B, nh, _CH, _NPAD))
    ts = (tk[:, :, :, 0].reshape(_B, nh, _K, _CH)
          .transpose(0, 1, 3, 2).reshape(_B, _C, _K))
    idx = (tk[:, :, :, 1].reshape(_B, nh, _K, _CH)
           .transpose(0, 1, 3, 2).reshape(_B, _C, _K)).astype(jnp.int32)
    tb = jnp.take_along_axis(boxes[:, None], idx[..., None], axis=2)

    nprog = _B // _G

    def to_kl(v):
        return (v.reshape(nprog, _G, _C, _K)
                 .transpose(0, 3, 1, 2)
                 .reshape(nprog, _K, 1, _L))

    s_kl = to_kl(ts)
    x1 = to_kl(tb[..., 0])
    y1 = to_kl(tb[..., 1])
    x2 = to_kl(tb[..., 2])
    y2 = to_kl(tb[..., 3])

    raw = _nms_select(s_kl, x1, y1, x2, y2)
    raw = raw.reshape(_B, _K, 128)
    out_scores = raw[:, :, 0]
    out_classes = raw[:, :, 1]
    out_boxes = raw[:, :, 2:6]
    valid = jnp.sum(out_scores > 0.0, axis=1).astype(jnp.int32)
    return out_boxes, out_scores, out_classes, valid
